# baseline (speedup 1.0000x reference)
import jax
import jax.numpy as jnp
from jax import lax
from jax.experimental import pallas as pl
from jax.experimental.pallas import tpu as pltpu
from jax.experimental.pallas import tpu_sc as plsc

_EOS = 8191
_NC = 2
_NS = 16
_L = 16


def _pool_body(hs_hbm, ids_hbm, mask_hbm, out_hbm,
               mask_v, part_v, all_v, ids_v, out_v, shared_s, sem):
    s = lax.axis_index("s")
    c = lax.axis_index("c")
    iota = lax.iota(jnp.int32, _L)

    rows_per_sub = mask_v.shape[0]
    pltpu.sync_copy(mask_hbm.at[pl.ds(s * rows_per_sub, rows_per_sub)], mask_v)
    acc = jnp.zeros((_L,), jnp.int32)
    for i in range(rows_per_sub):
        acc = acc + mask_v[i]
    part_v[0] = acc
    pltpu.sync_copy(part_v, shared_s.at[pl.ds(s, 1)])
    plsc.subcore_barrier()
    pltpu.sync_copy(shared_s, all_v)
    tot = jnp.zeros((_L,), jnp.int32)
    for i in range(_NS):
        tot = tot + all_v[i]
    sl = jnp.sum(tot) - 1

    row = lax.div(sl, _L)
    lane = lax.rem(sl, _L)
    rowp = lax.div(sl - 1, _L)
    lanep = lax.rem(sl - 1, _L)
    idx_ids = jnp.where(iota == 0, rowp, row)
    pltpu.async_copy(ids_hbm.at[idx_ids], ids_v, sem).wait()
    one = jnp.ones((_L,), jnp.int32)
    zero = jnp.zeros((_L,), jnp.int32)
    sel_cnt = jnp.sum(jnp.where((iota == lane) & (ids_v[1] == _EOS), one, zero))
    prev_cnt = jnp.sum(
        jnp.where((iota == lanep) & (ids_v[0] == _EOS), one, zero))
    valid = jnp.logical_and(sel_cnt == 1, prev_cnt == 0)

    rows_per_hidden = hs_hbm.shape[0] // mask_hbm.shape[0] // _L
    out_rows = out_v.shape[0]
    idx_hs = sl * rows_per_hidden + s * out_rows + iota
    pltpu.async_copy(hs_hbm.at[idx_hs], out_v, sem).wait()

    @pl.when(jnp.logical_not(valid))
    def _():
        nan_vec = jnp.full((_L,), jnp.nan, jnp.float32)
        for i in range(out_rows):
            out_v[i] = nan_vec

    @pl.when(c == 0)
    def _():
        pltpu.sync_copy(out_v, out_hbm.at[pl.ds(s * out_rows, out_rows)])


def kernel(hidden_states, input_ids, attention_mask):
    B, S, D = hidden_states.shape
    assert B == 1 and S % (_NS * _L) == 0 and D % (_NS * _L) == 0
    hs = hidden_states.reshape(S * D // _L, _L)
    ids = input_ids.reshape(S // _L, _L)
    mask = attention_mask.reshape(S // _L, _L)

    rows_per_sub = (S // _L) // _NS
    out_rows = D // _L // _NS

    call = pl.kernel(
        _pool_body,
        out_type=jax.ShapeDtypeStruct((D // _L, _L), jnp.float32),
        mesh=plsc.VectorSubcoreMesh(
            core_axis_name="c", subcore_axis_name="s",
            num_cores=_NC, num_subcores=_NS),
        scratch_types=[
            pltpu.VMEM((rows_per_sub, _L), jnp.int32),
            pltpu.VMEM((1, _L), jnp.int32),
            pltpu.VMEM((_NS, _L), jnp.int32),
            pltpu.VMEM((_L, _L), jnp.int32),
            pltpu.VMEM((out_rows, _L), jnp.float32),
            pltpu.VMEM_SHARED((_NS, _L), jnp.int32),
            pltpu.SemaphoreType.DMA,
        ],
        compiler_params=pltpu.CompilerParams(
            needs_layout_passes=False, use_tc_tiling_on_sc=False),
    )
    out = call(hs, ids, mask)
    return out.reshape(B, D)


# speedup vs baseline: 4.8978x; 4.8978x over previous
import jax
import jax.numpy as jnp
from jax import lax
from jax.experimental import pallas as pl
from jax.experimental.pallas import tpu as pltpu
from jax.experimental.pallas import tpu_sc as plsc

_EOS = 8191
_NC = 2
_NS = 16
_L = 16


def _pool_body(hs_hbm, ids_hbm, mask_hbm, out_hbm,
               mask_v, ids_v, slv_v, hs_v, sem):
    s = lax.axis_index("s")
    c = lax.axis_index("c")

    @pl.when(jnp.logical_and(s == 0, c == 0))
    def _():
        iota = lax.iota(jnp.int32, _L)
        S = mask_v.shape[1]

        pltpu.sync_copy(mask_hbm, mask_v)
        acc = jnp.zeros((_L,), jnp.int32)
        for i in range(S // _L):
            acc = acc + mask_v[0, pl.ds(i * _L, _L)]
        sl = jnp.sum(acc) - 1

        pltpu.sync_copy(ids_hbm, ids_v)
        base_s = lax.div(sl, _L) * _L
        base_p = lax.div(sl - 1, _L) * _L
        sel_vec = ids_v[0, pl.ds(base_s, _L)]
        prev_vec = ids_v[0, pl.ds(base_p, _L)]
        one = jnp.ones((_L,), jnp.int32)
        zero = jnp.zeros((_L,), jnp.int32)
        sel_cnt = jnp.sum(jnp.where(
            (iota == sl - base_s) & (sel_vec == _EOS), one, zero))
        prev_cnt = jnp.sum(jnp.where(
            (iota == sl - 1 - base_p) & (prev_vec == _EOS), one, zero))
        valid = jnp.logical_and(sel_cnt == 1, prev_cnt == 0)

        slv_v[0] = jnp.full((_L,), sl, jnp.int32)
        pltpu.async_copy(hs_hbm.at[slv_v.at[0, pl.ds(0, 1)]], hs_v, sem).wait()

        @pl.when(jnp.logical_not(valid))
        def _():
            nan_vec = jnp.full((_L,), jnp.nan, jnp.float32)
            for j in range(hs_v.shape[1] // _L):
                hs_v[0, pl.ds(j * _L, _L)] = nan_vec

        pltpu.sync_copy(hs_v, out_hbm)


def kernel(hidden_states, input_ids, attention_mask):
    B, S, D = hidden_states.shape
    assert B == 1 and S % _L == 0 and D % _L == 0
    hs = hidden_states.reshape(S, D)

    call = pl.kernel(
        _pool_body,
        out_type=jax.ShapeDtypeStruct((1, D), jnp.float32),
        mesh=plsc.VectorSubcoreMesh(
            core_axis_name="c", subcore_axis_name="s",
            num_cores=_NC, num_subcores=_NS),
        scratch_types=[
            pltpu.VMEM((1, S), jnp.int32),
            pltpu.VMEM((1, S), jnp.int32),
            pltpu.VMEM((1, _L), jnp.int32),
            pltpu.VMEM((1, D), jnp.float32),
            pltpu.SemaphoreType.DMA,
        ],
        compiler_params=pltpu.CompilerParams(needs_layout_passes=False),
    )
    return call(hs, input_ids, attention_mask)


# speedup vs baseline: 5.5240x; 1.1278x over previous
import jax
import jax.numpy as jnp
from jax import lax
from jax.experimental import pallas as pl
from jax.experimental.pallas import tpu as pltpu
from jax.experimental.pallas import tpu_sc as plsc

_EOS = 8191
_NC = 2
_NS = 16
_L = 16


def _pool_body(hs_hbm, ids_hbm, mask_hbm, out_hbm,
               mask_v, ids_v, slv_v, hs_v, sem, sem2):
    s = lax.axis_index("s")

    @pl.when(s == 0)
    def _():
        iota = lax.iota(jnp.int32, _L)
        S = mask_v.shape[1]

        mask_cp = pltpu.async_copy(mask_hbm, mask_v, sem)
        ids_cp = pltpu.async_copy(ids_hbm, ids_v, sem2)

        mask_cp.wait()
        acc = jnp.zeros((_L,), jnp.int32)
        for i in range(S // _L):
            acc = acc + mask_v[0, pl.ds(i * _L, _L)]
        sl = jnp.sum(acc) - 1

        ids_cp.wait()
        base_s = lax.div(sl, _L) * _L
        base_p = lax.div(sl - 1, _L) * _L
        sel_vec = ids_v[0, pl.ds(base_s, _L)]
        prev_vec = ids_v[0, pl.ds(base_p, _L)]
        one = jnp.ones((_L,), jnp.int32)
        zero = jnp.zeros((_L,), jnp.int32)
        sel_cnt = jnp.sum(jnp.where(
            (iota == sl - base_s) & (sel_vec == _EOS), one, zero))
        prev_cnt = jnp.sum(jnp.where(
            (iota == sl - 1 - base_p) & (prev_vec == _EOS), one, zero))
        valid = jnp.logical_and(sel_cnt == 1, prev_cnt == 0)

        slv_v[0] = jnp.full((_L,), sl, jnp.int32)
        pltpu.async_copy(hs_hbm.at[slv_v.at[0, pl.ds(0, 1)]], hs_v, sem).wait()

        @pl.when(jnp.logical_not(valid))
        def _():
            nan_vec = jnp.full((_L,), jnp.nan, jnp.float32)
            for j in range(hs_v.shape[1] // _L):
                hs_v[0, pl.ds(j * _L, _L)] = nan_vec

        pltpu.sync_copy(hs_v, out_hbm)


def kernel(hidden_states, input_ids, attention_mask):
    B, S, D = hidden_states.shape
    assert B == 1 and S % _L == 0 and D % _L == 0
    hs = hidden_states.reshape(S, D)

    call = pl.kernel(
        _pool_body,
        out_type=jax.ShapeDtypeStruct((1, D), jnp.float32),
        mesh=plsc.VectorSubcoreMesh(
            core_axis_name="c", subcore_axis_name="s",
            num_cores=1, num_subcores=_NS),
        scratch_types=[
            pltpu.VMEM((1, S), jnp.int32),
            pltpu.VMEM((1, S), jnp.int32),
            pltpu.VMEM((1, _L), jnp.int32),
            pltpu.VMEM((1, D), jnp.float32),
            pltpu.SemaphoreType.DMA,
            pltpu.SemaphoreType.DMA,
        ],
        compiler_params=pltpu.CompilerParams(needs_layout_passes=False),
    )
    return call(hs, input_ids, attention_mask)


# speedup vs baseline: 5.6692x; 1.0263x over previous
import jax
import jax.numpy as jnp
from jax import lax
from jax.experimental import pallas as pl
from jax.experimental.pallas import tpu as pltpu
from jax.experimental.pallas import tpu_sc as plsc

_EOS = 8191
_NS = 16
_L = 16
_W = 128


def _pool_body(hs_hbm, ids_hbm, mask_hbm, out_hbm,
               mask_v, ids_v, slv_v, hs_v, sem):
    s = lax.axis_index("s")

    @pl.when(s == 0)
    def _():
        iota = lax.iota(jnp.int32, _L)
        S = mask_v.shape[1]

        pltpu.sync_copy(mask_hbm, mask_v)

        def _sum_step(i, acc):
            for k in range(8):
                acc = acc + mask_v[0, pl.ds(i * 8 * _L + k * _L, _L)]
            return acc

        acc = lax.fori_loop(0, S // _L // 8, _sum_step,
                            jnp.zeros((_L,), jnp.int32))
        sl = jnp.sum(acc) - 1

        base_p = pl.multiple_of(lax.div(sl - 1, _W) * _W, _W)
        base_s = pl.multiple_of(lax.div(sl, _W) * _W, _W)
        pltpu.sync_copy(ids_hbm.at[pl.ds(0, 1), pl.ds(base_p, _W)],
                        ids_v.at[pl.ds(0, 1)])
        pltpu.sync_copy(ids_hbm.at[pl.ds(0, 1), pl.ds(base_s, _W)],
                        ids_v.at[pl.ds(1, 1)])
        rel_p = sl - 1 - base_p
        rel_s = sl - base_s
        sel_vec = ids_v[1, pl.ds(lax.div(rel_s, _L) * _L, _L)]
        prev_vec = ids_v[0, pl.ds(lax.div(rel_p, _L) * _L, _L)]
        one = jnp.ones((_L,), jnp.int32)
        zero = jnp.zeros((_L,), jnp.int32)
        sel_cnt = jnp.sum(jnp.where(
            (iota == lax.rem(rel_s, _L)) & (sel_vec == _EOS), one, zero))
        prev_cnt = jnp.sum(jnp.where(
            (iota == lax.rem(rel_p, _L)) & (prev_vec == _EOS), one, zero))
        valid = jnp.logical_and(sel_cnt == 1, prev_cnt == 0)

        slv_v[0] = jnp.full((_L,), sl, jnp.int32)
        pltpu.async_copy(hs_hbm.at[slv_v.at[0, pl.ds(0, 1)]], hs_v, sem).wait()

        @pl.when(jnp.logical_not(valid))
        def _():
            nan_vec = jnp.full((_L,), jnp.nan, jnp.float32)

            def _nan_step(j, _):
                hs_v[0, pl.ds(j * _L, _L)] = nan_vec
                return 0

            lax.fori_loop(0, hs_v.shape[1] // _L, _nan_step, 0)

        pltpu.sync_copy(hs_v, out_hbm)


def kernel(hidden_states, input_ids, attention_mask):
    B, S, D = hidden_states.shape
    assert B == 1 and S % (8 * _L) == 0 and D % _L == 0
    hs = hidden_states.reshape(S, D)

    call = pl.kernel(
        _pool_body,
        out_type=jax.ShapeDtypeStruct((1, D), jnp.float32),
        mesh=plsc.VectorSubcoreMesh(
            core_axis_name="c", subcore_axis_name="s",
            num_cores=1, num_subcores=_NS),
        scratch_types=[
            pltpu.VMEM((1, S), jnp.int32),
            pltpu.VMEM((2, _W), jnp.int32),
            pltpu.VMEM((1, _L), jnp.int32),
            pltpu.VMEM((1, D), jnp.float32),
            pltpu.SemaphoreType.DMA,
        ],
        compiler_params=pltpu.CompilerParams(needs_layout_passes=False),
    )
    return call(hs, input_ids, attention_mask)


# speedup vs baseline: 5.9077x; 1.0421x over previous
import jax
import jax.numpy as jnp
from jax import lax
from jax.experimental import pallas as pl
from jax.experimental.pallas import tpu as pltpu
from jax.experimental.pallas import tpu_sc as plsc

_EOS = 8191
_NS = 16
_L = 16
_W = 128


def _pool_body(hs_hbm, ids_hbm, mask_hbm, out_hbm,
               mask_v, ids_v, slv_v, hs_v, sem):
    s = lax.axis_index("s")

    @pl.when(s == 0)
    def _():
        iota = lax.iota(jnp.int32, _L)
        S = mask_v.shape[1]

        pltpu.sync_copy(mask_hbm, mask_v)

        def _sum_step(i, acc):
            for k in range(8):
                acc = acc + mask_v[0, pl.ds(i * 8 * _L + k * _L, _L)]
            return acc

        acc = lax.fori_loop(0, S // _L // 8, _sum_step,
                            jnp.zeros((_L,), jnp.int32))
        sl = jnp.sum(acc) - 1

        slv_v[0] = jnp.full((_L,), sl, jnp.int32)
        row_cp = pltpu.async_copy(hs_hbm.at[slv_v.at[0, pl.ds(0, 1)]],
                                  hs_v, sem)

        base_p = pl.multiple_of(lax.div(sl - 1, _W) * _W, _W)
        base_s = pl.multiple_of(lax.div(sl, _W) * _W, _W)
        pltpu.sync_copy(ids_hbm.at[pl.ds(0, 1), pl.ds(base_p, _W)],
                        ids_v.at[pl.ds(0, 1)])
        pltpu.sync_copy(ids_hbm.at[pl.ds(0, 1), pl.ds(base_s, _W)],
                        ids_v.at[pl.ds(1, 1)])
        rel_p = sl - 1 - base_p
        rel_s = sl - base_s
        sel_vec = ids_v[1, pl.ds(lax.div(rel_s, _L) * _L, _L)]
        prev_vec = ids_v[0, pl.ds(lax.div(rel_p, _L) * _L, _L)]
        one = jnp.ones((_L,), jnp.int32)
        zero = jnp.zeros((_L,), jnp.int32)
        sel_cnt = jnp.sum(jnp.where(
            (iota == lax.rem(rel_s, _L)) & (sel_vec == _EOS), one, zero))
        prev_cnt = jnp.sum(jnp.where(
            (iota == lax.rem(rel_p, _L)) & (prev_vec == _EOS), one, zero))
        valid = jnp.logical_and(sel_cnt == 1, prev_cnt == 0)

        row_cp.wait()

        @pl.when(jnp.logical_not(valid))
        def _():
            nan_vec = jnp.full((_L,), jnp.nan, jnp.float32)

            def _nan_step(j, _):
                hs_v[0, pl.ds(j * _L, _L)] = nan_vec
                return 0

            lax.fori_loop(0, hs_v.shape[1] // _L, _nan_step, 0)

        pltpu.sync_copy(hs_v, out_hbm)


def kernel(hidden_states, input_ids, attention_mask):
    B, S, D = hidden_states.shape
    assert B == 1 and S % (8 * _L) == 0 and D % _L == 0
    hs = hidden_states.reshape(S, D)

    call = pl.kernel(
        _pool_body,
        out_type=jax.ShapeDtypeStruct((1, D), jnp.float32),
        mesh=plsc.VectorSubcoreMesh(
            core_axis_name="c", subcore_axis_name="s",
            num_cores=1, num_subcores=_NS),
        scratch_types=[
            pltpu.VMEM((1, S), jnp.int32),
            pltpu.VMEM((2, _W), jnp.int32),
            pltpu.VMEM((1, _L), jnp.int32),
            pltpu.VMEM((1, D), jnp.float32),
            pltpu.SemaphoreType.DMA,
        ],
        compiler_params=pltpu.CompilerParams(
            needs_layout_passes=False, skip_device_barrier=True),
    )
    return call(hs, input_ids, attention_mask)
